# Initial kernel scaffold; baseline (speedup 1.0000x reference)
import jax
import jax.numpy as jnp
from jax.experimental import pallas as pl


def kernel(input, wg, w1, w2):
    raise NotImplementedError("write your pallas kernel here")


# speedup vs baseline: 2.2397x; 2.2397x over previous
import functools

import jax
import jax.numpy as jnp
from jax import lax
from jax.experimental import pallas as pl
from jax.experimental.pallas import tpu as pltpu
from jax.experimental.pallas import tpu_sc as plsc

S = 4096
M = 768
E = 16
DFF = 2048
TOPK = 2
CAP = TOPK * ((S + E - 1) // E)
NROWS = E * CAP
DUMP = NROWS
NPAD = NROWS + 8

NC = 2
NS = 16
NW = NC * NS
TPW = S // NW


def _cumsum0(a):
    k = 1
    n = a.shape[0]
    while k < n:
        a = a + jnp.concatenate(
            [jnp.zeros((k, a.shape[1]), a.dtype), a[:-k]], axis=0)
        k *= 2
    return a


def _routing_body(x_ref, wgt_ref, fd1_ref, fd2_ref, fc1_ref, fc2_ref,
                  g1_ref, g2_ref, nv_ref, loss_ref):
    x = x_ref[...]
    wgt = wgt_ref[...]
    logits = jnp.dot(x, wgt, preferred_element_type=jnp.float32)
    ids = lax.broadcasted_iota(jnp.int32, (S, E), 1)
    mx = jnp.max(logits, axis=1, keepdims=True)
    ex = jnp.exp(logits - mx)
    gates = ex / jnp.sum(ex, axis=1, keepdims=True)
    e1 = jnp.min(jnp.where(logits == mx, ids, E), axis=1, keepdims=True)
    mask1 = ids == e1
    neg = jnp.where(mask1, -jnp.inf, logits)
    mx2 = jnp.max(neg, axis=1, keepdims=True)
    e2 = jnp.min(jnp.where(neg == mx2, ids, E), axis=1, keepdims=True)
    mask2 = ids == e2
    m1f = mask1.astype(jnp.float32)
    m2f = mask2.astype(jnp.float32)
    g1 = jnp.sum(gates * m1f, axis=1, keepdims=True)
    g2 = jnp.sum(gates * m2f, axis=1, keepdims=True)
    me = jnp.sum(gates, axis=0, keepdims=True)
    ce = jnp.sum(m1f, axis=0, keepdims=True)
    loss_ref[...] = (jnp.sum(me * ce) * (E / (S * S))).reshape(1, 1)
    cum1 = _cumsum0(m1f)
    cum2 = _cumsum0(m2f)
    c2tot = jnp.sum(m2f, axis=0, keepdims=True)
    loc1 = jnp.sum((cum1 - 1.0) * m1f, axis=1, keepdims=True)
    loc2 = jnp.sum((cum2 - 1.0 + ce) * m2f, axis=1, keepdims=True)
    nv_ref[...] = jnp.minimum(ce + c2tot, float(CAP)).astype(jnp.int32)
    loc1i = loc1.astype(jnp.int32)
    loc2i = loc2.astype(jnp.int32)
    v1 = loc1i < CAP
    v2 = loc2i < CAP
    flat1 = e1 * CAP + loc1i
    flat2 = e2 * CAP + loc2i
    fd1_ref[...] = jnp.where(v1, flat1, DUMP)
    fd2_ref[...] = jnp.where(v2, flat2, DUMP)
    fc1_ref[...] = jnp.where(v1, flat1, 0)
    fc2_ref[...] = jnp.where(v2, flat2, 0)
    denom = jnp.maximum(g1 + g2, jnp.finfo(jnp.float32).eps)
    g1_ref[...] = jnp.where(v1, g1 / denom, 0.0)
    g2_ref[...] = jnp.where(v2, g2 / denom, 0.0)


_routing = pl.pallas_call(
    _routing_body,
    out_shape=(
        jax.ShapeDtypeStruct((S, 1), jnp.int32),
        jax.ShapeDtypeStruct((S, 1), jnp.int32),
        jax.ShapeDtypeStruct((S, 1), jnp.int32),
        jax.ShapeDtypeStruct((S, 1), jnp.int32),
        jax.ShapeDtypeStruct((S, 1), jnp.float32),
        jax.ShapeDtypeStruct((S, 1), jnp.float32),
        jax.ShapeDtypeStruct((1, E), jnp.int32),
        jax.ShapeDtypeStruct((1, 1), jnp.float32),
    ),
)


@functools.cache
def _get_dispatch():
    mesh = plsc.VectorSubcoreMesh(core_axis_name="c", subcore_axis_name="s")

    @functools.partial(
        pl.kernel,
        out_type=jax.ShapeDtypeStruct((NPAD, M), jnp.float32),
        mesh=mesh,
        scratch_types=[
            pltpu.VMEM((TPW,), jnp.int32),
            pltpu.VMEM((TPW,), jnp.int32),
            pltpu.VMEM((TPW, M), jnp.float32),
            pltpu.SemaphoreType.DMA,
            pltpu.SemaphoreType.DMA,
        ],
    )
    def _dispatch(x_hbm, f1_hbm, f2_hbm, out_hbm, i1_v, i2_v, rows_v, s1, s2):
        wid = lax.axis_index("s") * NC + lax.axis_index("c")
        base = wid * TPW
        pltpu.sync_copy(f1_hbm.at[pl.ds(base, TPW)], i1_v)
        pltpu.sync_copy(f2_hbm.at[pl.ds(base, TPW)], i2_v)
        pltpu.sync_copy(x_hbm.at[pl.ds(base, TPW)], rows_v)
        c1 = pltpu.async_copy(rows_v, out_hbm.at[i1_v], s1)
        c2 = pltpu.async_copy(rows_v, out_hbm.at[i2_v], s2)
        c1.wait()
        c2.wait()

    return _dispatch


def _ffn_body(nv_ref, d_ref, w1_ref, w2_ref, o_ref):
    e = pl.program_id(0)
    n = nv_ref[0, e]
    rows = lax.broadcasted_iota(jnp.int32, (CAP, 1), 0)
    xb = jnp.where(rows < n, d_ref[...], 0.0)
    h = jnp.maximum(
        jnp.dot(xb, w1_ref[0], preferred_element_type=jnp.float32), 0.0)
    o_ref[...] = jnp.dot(h, w2_ref[0], preferred_element_type=jnp.float32)


_ffn = pl.pallas_call(
    _ffn_body,
    grid=(E,),
    in_specs=[
        pl.BlockSpec(memory_space=pltpu.SMEM),
        pl.BlockSpec((CAP, M), lambda e: (e, 0)),
        pl.BlockSpec((1, M, DFF), lambda e: (e, 0, 0)),
        pl.BlockSpec((1, DFF, M), lambda e: (e, 0, 0)),
    ],
    out_specs=pl.BlockSpec((CAP, M), lambda e: (e, 0)),
    out_shape=jax.ShapeDtypeStruct((NROWS, M), jnp.float32),
)


@functools.cache
def _get_combine():
    mesh = plsc.VectorSubcoreMesh(core_axis_name="c", subcore_axis_name="s")

    @functools.partial(
        pl.kernel,
        out_type=(
            jax.ShapeDtypeStruct((S, M), jnp.float32),
            jax.ShapeDtypeStruct((S, M), jnp.float32),
        ),
        mesh=mesh,
        scratch_types=[
            pltpu.VMEM((TPW,), jnp.int32),
            pltpu.VMEM((TPW, M), jnp.float32),
            pltpu.SemaphoreType.DMA,
        ],
    )
    def _combine(out_hbm, f1_hbm, f2_hbm, r1_hbm, r2_hbm, i_v, rows_v, sem):
        wid = lax.axis_index("s") * NC + lax.axis_index("c")
        base = wid * TPW
        pltpu.sync_copy(f1_hbm.at[pl.ds(base, TPW)], i_v)
        pltpu.async_copy(out_hbm.at[i_v], rows_v, sem).wait()
        pltpu.sync_copy(rows_v, r1_hbm.at[pl.ds(base, TPW)])
        pltpu.sync_copy(f2_hbm.at[pl.ds(base, TPW)], i_v)
        pltpu.async_copy(out_hbm.at[i_v], rows_v, sem).wait()
        pltpu.sync_copy(rows_v, r2_hbm.at[pl.ds(base, TPW)])

    return _combine


_WB = 512


def _wsum_body(r1_ref, r2_ref, g1_ref, g2_ref, y_ref):
    y_ref[...] = g1_ref[...] * r1_ref[...] + g2_ref[...] * r2_ref[...]


_wsum = pl.pallas_call(
    _wsum_body,
    grid=(S // _WB,),
    in_specs=[
        pl.BlockSpec((_WB, M), lambda i: (i, 0)),
        pl.BlockSpec((_WB, M), lambda i: (i, 0)),
        pl.BlockSpec((_WB, 1), lambda i: (i, 0)),
        pl.BlockSpec((_WB, 1), lambda i: (i, 0)),
    ],
    out_specs=pl.BlockSpec((_WB, M), lambda i: (i, 0)),
    out_shape=jax.ShapeDtypeStruct((S, M), jnp.float32),
)


def kernel(input, wg, w1, w2):
    x = input
    wgt = wg.T
    fd1, fd2, fc1, fc2, g1, g2, nv, loss = _routing(x, wgt)
    disp = _get_dispatch()(x, fd1.reshape(S), fd2.reshape(S))
    out = _ffn(nv, disp[:NROWS], w1, w2)
    r1, r2 = _get_combine()(out, fc1.reshape(S), fc2.reshape(S))
    y = _wsum(r1, r2, g1, g2)
    return y, loss.reshape(())


# speedup vs baseline: 2.3895x; 1.0669x over previous
import functools

import jax
import jax.numpy as jnp
from jax import lax
from jax.experimental import pallas as pl
from jax.experimental.pallas import tpu as pltpu
from jax.experimental.pallas import tpu_sc as plsc

S = 4096
M = 768
E = 16
DFF = 2048
TOPK = 2
CAP = TOPK * ((S + E - 1) // E)
NROWS = E * CAP
DUMP = NROWS
NPAD = NROWS + 8

NC = 2
NS = 16
NW = NC * NS
TPW = S // NW


def _cumsum0(a):
    k = 1
    n = a.shape[0]
    while k < n:
        a = a + jnp.concatenate(
            [jnp.zeros((k, a.shape[1]), a.dtype), a[:-k]], axis=0)
        k *= 2
    return a


def _routing_body(x_ref, wgt_ref, fd1_ref, fd2_ref, fc1_ref, fc2_ref,
                  g1_ref, g2_ref, nv_ref, loss_ref):
    x = x_ref[...]
    wgt = wgt_ref[...]
    logits = jnp.dot(x, wgt, preferred_element_type=jnp.float32)
    ids = lax.broadcasted_iota(jnp.int32, (S, E), 1)
    mx = jnp.max(logits, axis=1, keepdims=True)
    ex = jnp.exp(logits - mx)
    gates = ex / jnp.sum(ex, axis=1, keepdims=True)
    e1 = jnp.min(jnp.where(logits == mx, ids, E), axis=1, keepdims=True)
    mask1 = ids == e1
    neg = jnp.where(mask1, -jnp.inf, logits)
    mx2 = jnp.max(neg, axis=1, keepdims=True)
    e2 = jnp.min(jnp.where(neg == mx2, ids, E), axis=1, keepdims=True)
    mask2 = ids == e2
    m1f = mask1.astype(jnp.float32)
    m2f = mask2.astype(jnp.float32)
    g1 = jnp.sum(gates * m1f, axis=1, keepdims=True)
    g2 = jnp.sum(gates * m2f, axis=1, keepdims=True)
    me = jnp.sum(gates, axis=0, keepdims=True)
    ce = jnp.sum(m1f, axis=0, keepdims=True)
    loss_ref[...] = (jnp.sum(me * ce) * (E / (S * S))).reshape(1, 1)
    cum1 = _cumsum0(m1f)
    cum2 = _cumsum0(m2f)
    c2tot = jnp.sum(m2f, axis=0, keepdims=True)
    loc1 = jnp.sum((cum1 - 1.0) * m1f, axis=1, keepdims=True)
    loc2 = jnp.sum((cum2 - 1.0 + ce) * m2f, axis=1, keepdims=True)
    nv_ref[...] = jnp.minimum(ce + c2tot, float(CAP)).astype(jnp.int32)
    loc1i = loc1.astype(jnp.int32)
    loc2i = loc2.astype(jnp.int32)
    v1 = loc1i < CAP
    v2 = loc2i < CAP
    flat1 = e1 * CAP + loc1i
    flat2 = e2 * CAP + loc2i
    fd1_ref[...] = jnp.where(v1, flat1, DUMP)
    fd2_ref[...] = jnp.where(v2, flat2, DUMP)
    fc1_ref[...] = jnp.where(v1, flat1, 0)
    fc2_ref[...] = jnp.where(v2, flat2, 0)
    denom = jnp.maximum(g1 + g2, jnp.finfo(jnp.float32).eps)
    g1_ref[...] = jnp.broadcast_to(jnp.where(v1, g1 / denom, 0.0), (S, E))
    g2_ref[...] = jnp.broadcast_to(jnp.where(v2, g2 / denom, 0.0), (S, E))


_routing = pl.pallas_call(
    _routing_body,
    out_shape=(
        jax.ShapeDtypeStruct((S, 1), jnp.int32),
        jax.ShapeDtypeStruct((S, 1), jnp.int32),
        jax.ShapeDtypeStruct((S, 1), jnp.int32),
        jax.ShapeDtypeStruct((S, 1), jnp.int32),
        jax.ShapeDtypeStruct((S, E), jnp.float32),
        jax.ShapeDtypeStruct((S, E), jnp.float32),
        jax.ShapeDtypeStruct((1, E), jnp.int32),
        jax.ShapeDtypeStruct((1, 1), jnp.float32),
    ),
)


@functools.cache
def _get_dispatch():
    mesh = plsc.VectorSubcoreMesh(core_axis_name="c", subcore_axis_name="s")

    @functools.partial(
        pl.kernel,
        out_type=jax.ShapeDtypeStruct((NPAD, M), jnp.float32),
        mesh=mesh,
        scratch_types=[
            pltpu.VMEM((TPW,), jnp.int32),
            pltpu.VMEM((TPW,), jnp.int32),
            pltpu.VMEM((TPW, M), jnp.float32),
            pltpu.SemaphoreType.DMA,
            pltpu.SemaphoreType.DMA,
        ],
    )
    def _dispatch(x_hbm, f1_hbm, f2_hbm, out_hbm, i1_v, i2_v, rows_v, s1, s2):
        wid = lax.axis_index("s") * NC + lax.axis_index("c")
        base = wid * TPW
        pltpu.sync_copy(f1_hbm.at[pl.ds(base, TPW)], i1_v)
        pltpu.sync_copy(f2_hbm.at[pl.ds(base, TPW)], i2_v)
        pltpu.sync_copy(x_hbm.at[pl.ds(base, TPW)], rows_v)
        c1 = pltpu.async_copy(rows_v, out_hbm.at[i1_v], s1)
        c2 = pltpu.async_copy(rows_v, out_hbm.at[i2_v], s2)
        c1.wait()
        c2.wait()

    return _dispatch


def _ffn_body(nv_ref, d_ref, w1_ref, w2_ref, o_ref):
    e = pl.program_id(0)
    n = nv_ref[0, e]
    rows = lax.broadcasted_iota(jnp.int32, (CAP, 1), 0)
    xb = jnp.where(rows < n, d_ref[...], 0.0)
    h = jnp.maximum(
        jnp.dot(xb, w1_ref[0], preferred_element_type=jnp.float32), 0.0)
    o_ref[...] = jnp.dot(h, w2_ref[0], preferred_element_type=jnp.float32)


_ffn = pl.pallas_call(
    _ffn_body,
    grid=(E,),
    in_specs=[
        pl.BlockSpec(memory_space=pltpu.SMEM),
        pl.BlockSpec((CAP, M), lambda e: (e, 0)),
        pl.BlockSpec((1, M, DFF), lambda e: (e, 0, 0)),
        pl.BlockSpec((1, DFF, M), lambda e: (e, 0, 0)),
    ],
    out_specs=pl.BlockSpec((CAP, M), lambda e: (e, 0)),
    out_shape=jax.ShapeDtypeStruct((NROWS, M), jnp.float32),
)


_CH = 64


@functools.cache
def _get_combine():
    mesh = plsc.VectorSubcoreMesh(core_axis_name="c", subcore_axis_name="s")

    @functools.partial(
        pl.kernel,
        out_type=jax.ShapeDtypeStruct((S, M), jnp.float32),
        mesh=mesh,
        scratch_types=[
            pltpu.VMEM((TPW,), jnp.int32),
            pltpu.VMEM((TPW,), jnp.int32),
            pltpu.VMEM((TPW * E,), jnp.float32),
            pltpu.VMEM((TPW * E,), jnp.float32),
            pltpu.VMEM((_CH, M), jnp.float32),
            pltpu.VMEM((_CH, M), jnp.float32),
            pltpu.SemaphoreType.DMA,
            pltpu.SemaphoreType.DMA,
        ],
    )
    def _combine(out_hbm, f1_hbm, f2_hbm, gg1_hbm, gg2_hbm, y_hbm,
                 iv1, iv2, ga, gb, a_v, b_v, sa, sb):
        wid = lax.axis_index("s") * NC + lax.axis_index("c")
        base = wid * TPW
        pltpu.sync_copy(f1_hbm.at[pl.ds(base, TPW)], iv1)
        pltpu.sync_copy(f2_hbm.at[pl.ds(base, TPW)], iv2)
        pltpu.sync_copy(gg1_hbm.at[pl.ds(base * E, TPW * E)], ga)
        pltpu.sync_copy(gg2_hbm.at[pl.ds(base * E, TPW * E)], gb)
        for c in range(TPW // _CH):
            ca = pltpu.async_copy(out_hbm.at[iv1.at[pl.ds(c * _CH, _CH)]],
                                  a_v, sa)
            cb = pltpu.async_copy(out_hbm.at[iv2.at[pl.ds(c * _CH, _CH)]],
                                  b_v, sb)
            ca.wait()
            cb.wait()

            def tok_body(t, carry, c=c):
                g1v = ga[pl.ds((c * _CH + t) * E, 16)]
                g2v = gb[pl.ds((c * _CH + t) * E, 16)]
                for v in range(M // 16):
                    sl = pl.ds(v * 16, 16)
                    a_v[t, sl] = g1v * a_v[t, sl] + g2v * b_v[t, sl]
                return carry

            lax.fori_loop(0, _CH, tok_body, 0)
            pltpu.sync_copy(a_v, y_hbm.at[pl.ds(base + c * _CH, _CH)])

    return _combine


def kernel(input, wg, w1, w2):
    x = input
    wgt = wg.T
    fd1, fd2, fc1, fc2, g1, g2, nv, loss = _routing(x, wgt)
    disp = _get_dispatch()(x, fd1.reshape(S), fd2.reshape(S))
    out = _ffn(nv, disp[:NROWS], w1, w2)
    y = _get_combine()(out, fc1.reshape(S), fc2.reshape(S),
                       g1.reshape(S * E), g2.reshape(S * E))
    return y, loss.reshape(())
